# Initial kernel scaffold; baseline (speedup 1.0000x reference)
import jax
import jax.numpy as jnp
from jax.experimental import pallas as pl


def kernel(xyz1, xyz2, points1, points2, w0, b0, g0, be0, w1, b1, g1, be1):
    raise NotImplementedError("write your pallas kernel here")


# speedup vs baseline: 21.6686x; 21.6686x over previous
import functools

import jax
import jax.numpy as jnp
from jax import lax
from jax.experimental import pallas as pl


_HI = jax.lax.Precision.HIGHEST


def _stage1_body(nt_inv, x1_ref, x2t_ref, p1_ref, p2_ref, w0_ref, b0_ref,
                 u_ref, s_ref, ss_ref):
    b = pl.program_id(0)
    t = pl.program_id(1)
    x1 = x1_ref[0]
    x2t = x2t_ref[0]
    S = x2t.shape[0]
    TN = x1.shape[1]

    rb = lambda v: v.astype(jnp.bfloat16).astype(jnp.float32)
    cross = (rb(x2t[:, 0:1]) * rb(x1[0:1, :])
             + rb(x2t[:, 1:2]) * rb(x1[1:2, :])
             + rb(x2t[:, 2:3]) * rb(x1[2:3, :]))
    n1 = (x1[0:1, :] * x1[0:1, :] + x1[1:2, :] * x1[1:2, :]
          + x1[2:3, :] * x1[2:3, :])
    n2 = (x2t[:, 0:1] * x2t[:, 0:1] + x2t[:, 1:2] * x2t[:, 1:2]
          + x2t[:, 2:3] * x2t[:, 2:3])
    d = -2.0 * cross + n1 + n2

    iota = lax.broadcasted_iota(jnp.int32, (S, TN), 0)
    dmins = []
    idxs = []
    for k in range(3):
        m = jnp.min(d, axis=0, keepdims=True)
        ik = jnp.min(jnp.where(d == m, iota, S), axis=0, keepdims=True)
        dmins.append(m)
        idxs.append(ik)
        if k < 2:
            d = jnp.where(iota == ik, jnp.float32(jnp.inf), d)

    r0 = 1.0 / (dmins[0] + 1e-8)
    r1 = 1.0 / (dmins[1] + 1e-8)
    r2 = 1.0 / (dmins[2] + 1e-8)
    norm = r0 + r1 + r2

    zero = jnp.zeros((S, TN), jnp.float32)
    wt = (jnp.where(iota == idxs[0], r0 / norm, zero)
          + jnp.where(iota == idxs[1], r1 / norm, zero)
          + jnp.where(iota == idxs[2], r2 / norm, zero))

    interp = lax.dot_general(p2_ref[0], wt, (((1,), (0,)), ((), ())),
                             preferred_element_type=jnp.float32,
                             precision=_HI)
    TNw = x1.shape[1]
    cat = jnp.concatenate(
        [p1_ref[0], interp,
         jnp.zeros((w0_ref.shape[1] - p1_ref.shape[1] - interp.shape[0], TNw),
                   jnp.float32)], axis=0)
    u = lax.dot_general(w0_ref[...].astype(jnp.bfloat16),
                        cat.astype(jnp.bfloat16), (((1,), (0,)), ((), ())),
                        preferred_element_type=jnp.float32) + b0_ref[...]
    u_ref[0] = u

    @pl.when(jnp.logical_and(b == 0, t == 0))
    def _():
        s_ref[...] = jnp.zeros_like(s_ref)
        ss_ref[...] = jnp.zeros_like(ss_ref)

    s_ref[...] += jnp.sum(u, axis=1, keepdims=True)
    ss_ref[...] += jnp.sum(u * u, axis=1, keepdims=True)


def _stage2_body(nt_inv, u_ref, s_ref, ss_ref, g_ref, be_ref, w1_ref, b1_ref,
                 v_ref, s1_ref, ss1_ref):
    b = pl.program_id(0)
    t = pl.program_id(1)
    mean = s_ref[...] * nt_inv
    var = ss_ref[...] * nt_inv - mean * mean
    sc = g_ref[...] / jnp.sqrt(var + 1e-5)
    sh = be_ref[...] - sc * mean
    r = jnp.maximum(sc * u_ref[0] + sh, 0.0)
    v = lax.dot_general(w1_ref[...].astype(jnp.bfloat16),
                        r.astype(jnp.bfloat16), (((1,), (0,)), ((), ())),
                        preferred_element_type=jnp.float32) + b1_ref[...]
    v_ref[0] = v

    @pl.when(jnp.logical_and(b == 0, t == 0))
    def _():
        s1_ref[...] = jnp.zeros_like(s1_ref)
        ss1_ref[...] = jnp.zeros_like(ss1_ref)

    s1_ref[...] += jnp.sum(v, axis=1, keepdims=True)
    ss1_ref[...] += jnp.sum(v * v, axis=1, keepdims=True)


def _stage3_body(nt_inv, v_ref, s_ref, ss_ref, g_ref, be_ref, o_ref):
    mean = s_ref[...] * nt_inv
    var = ss_ref[...] * nt_inv - mean * mean
    sc = g_ref[...] / jnp.sqrt(var + 1e-5)
    sh = be_ref[...] - sc * mean
    o_ref[0] = jnp.maximum(sc * v_ref[0] + sh, 0.0)


def kernel(xyz1, xyz2, points1, points2, w0, b0, g0, be0, w1, b1, g1, be1):
    B, _, N = xyz1.shape
    S = xyz2.shape[2]
    D1 = points1.shape[1]
    D2 = points2.shape[1]
    C0 = w0.shape[0]
    C1 = w1.shape[0]
    nt_inv = 1.0 / (B * N)

    x1p = jnp.pad(xyz1, ((0, 0), (0, 5), (0, 0)))
    x2t = jnp.pad(jnp.swapaxes(xyz2, 1, 2),
                  ((0, 0), (0, 0), (0, 5)))
    KC = 256
    w0p = jnp.pad(w0, ((0, 0), (0, KC - (D1 + D2))))
    col = lambda a: a.reshape(-1, 1)

    TN = 256
    grid = (B, N // TN)
    u, s0, ss0 = pl.pallas_call(
        functools.partial(_stage1_body, nt_inv),
        grid=grid,
        in_specs=[
            pl.BlockSpec((1, 8, TN), lambda b, t: (b, 0, t)),
            pl.BlockSpec((1, S, 8), lambda b, t: (b, 0, 0)),
            pl.BlockSpec((1, D1, TN), lambda b, t: (b, 0, t)),
            pl.BlockSpec((1, D2, S), lambda b, t: (b, 0, 0)),
            pl.BlockSpec((C0, KC), lambda b, t: (0, 0)),
            pl.BlockSpec((C0, 1), lambda b, t: (0, 0)),
        ],
        out_specs=[
            pl.BlockSpec((1, C0, TN), lambda b, t: (b, 0, t)),
            pl.BlockSpec((C0, 1), lambda b, t: (0, 0)),
            pl.BlockSpec((C0, 1), lambda b, t: (0, 0)),
        ],
        out_shape=[
            jax.ShapeDtypeStruct((B, C0, N), jnp.float32),
            jax.ShapeDtypeStruct((C0, 1), jnp.float32),
            jax.ShapeDtypeStruct((C0, 1), jnp.float32),
        ],
    )(x1p, x2t, points1, points2, w0p, col(b0))

    TN2 = 512
    v, s1, ss1 = pl.pallas_call(
        functools.partial(_stage2_body, nt_inv),
        grid=(B, N // TN2),
        in_specs=[
            pl.BlockSpec((1, C0, TN2), lambda b, t: (b, 0, t)),
            pl.BlockSpec((C0, 1), lambda b, t: (0, 0)),
            pl.BlockSpec((C0, 1), lambda b, t: (0, 0)),
            pl.BlockSpec((C0, 1), lambda b, t: (0, 0)),
            pl.BlockSpec((C0, 1), lambda b, t: (0, 0)),
            pl.BlockSpec((C1, C0), lambda b, t: (0, 0)),
            pl.BlockSpec((C1, 1), lambda b, t: (0, 0)),
        ],
        out_specs=[
            pl.BlockSpec((1, C1, TN2), lambda b, t: (b, 0, t)),
            pl.BlockSpec((C1, 1), lambda b, t: (0, 0)),
            pl.BlockSpec((C1, 1), lambda b, t: (0, 0)),
        ],
        out_shape=[
            jax.ShapeDtypeStruct((B, C1, N), jnp.float32),
            jax.ShapeDtypeStruct((C1, 1), jnp.float32),
            jax.ShapeDtypeStruct((C1, 1), jnp.float32),
        ],
    )(u, s0, ss0, col(g0), col(be0), w1, col(b1))

    TN3 = 2048
    out = pl.pallas_call(
        functools.partial(_stage3_body, nt_inv),
        grid=(B, N // TN3),
        in_specs=[
            pl.BlockSpec((1, C1, TN3), lambda b, t: (b, 0, t)),
            pl.BlockSpec((C1, 1), lambda b, t: (0, 0)),
            pl.BlockSpec((C1, 1), lambda b, t: (0, 0)),
            pl.BlockSpec((C1, 1), lambda b, t: (0, 0)),
            pl.BlockSpec((C1, 1), lambda b, t: (0, 0)),
        ],
        out_specs=pl.BlockSpec((1, C1, TN3), lambda b, t: (b, 0, t)),
        out_shape=jax.ShapeDtypeStruct((B, C1, N), jnp.float32),
    )(v, s1, ss1, col(g1), col(be1))
    return out


# speedup vs baseline: 24.1918x; 1.1164x over previous
import functools

import jax
import jax.numpy as jnp
from jax import lax
from jax.experimental import pallas as pl


_HI = jax.lax.Precision.HIGHEST


def _stage1_body(nt_inv, x1_ref, x2t_ref, p1_ref, p2_ref, w0_ref, b0_ref,
                 u_ref, s_ref, ss_ref):
    b = pl.program_id(0)
    t = pl.program_id(1)
    x1 = x1_ref[0]
    x2t = x2t_ref[0]
    S = x2t.shape[0]
    TN = x1.shape[1]

    rb = lambda v: v.astype(jnp.bfloat16).astype(jnp.float32)
    cross = (rb(x2t[:, 0:1]) * rb(x1[0:1, :])
             + rb(x2t[:, 1:2]) * rb(x1[1:2, :])
             + rb(x2t[:, 2:3]) * rb(x1[2:3, :]))
    n1 = (x1[0:1, :] * x1[0:1, :] + x1[1:2, :] * x1[1:2, :]
          + x1[2:3, :] * x1[2:3, :])
    n2 = (x2t[:, 0:1] * x2t[:, 0:1] + x2t[:, 1:2] * x2t[:, 1:2]
          + x2t[:, 2:3] * x2t[:, 2:3])
    d = -2.0 * cross + n1 + n2

    iota = lax.broadcasted_iota(jnp.int32, (S, TN), 0)
    dmins = []
    eqs = []
    for k in range(3):
        m = jnp.min(d, axis=0, keepdims=True)
        ik = jnp.min(jnp.where(d == m, iota, S), axis=0, keepdims=True)
        eq = iota == ik
        dmins.append(m)
        eqs.append(eq)
        if k < 2:
            d = jnp.where(eq, jnp.float32(jnp.inf), d)

    r0 = 1.0 / (dmins[0] + 1e-8)
    r1 = 1.0 / (dmins[1] + 1e-8)
    r2 = 1.0 / (dmins[2] + 1e-8)
    norm = r0 + r1 + r2

    zero = jnp.zeros((S, TN), jnp.float32)
    wt = (jnp.where(eqs[0], r0 / norm, zero)
          + jnp.where(eqs[1], r1 / norm, zero)
          + jnp.where(eqs[2], r2 / norm, zero))

    p2 = p2_ref[0]
    p2h = p2.astype(jnp.bfloat16)
    p2l = (p2 - p2h.astype(jnp.float32)).astype(jnp.bfloat16)
    wth = wt.astype(jnp.bfloat16)
    wtl = (wt - wth.astype(jnp.float32)).astype(jnp.bfloat16)
    dn = (((1,), (0,)), ((), ()))
    f32 = jnp.float32
    interp = (lax.dot_general(p2h, wth, dn, preferred_element_type=f32)
              + lax.dot_general(p2h, wtl, dn, preferred_element_type=f32)
              + lax.dot_general(p2l, wth, dn, preferred_element_type=f32))
    TNw = x1.shape[1]
    cat = jnp.concatenate(
        [p1_ref[0], interp,
         jnp.zeros((w0_ref.shape[1] - p1_ref.shape[1] - interp.shape[0], TNw),
                   jnp.float32)], axis=0)
    u = lax.dot_general(w0_ref[...].astype(jnp.bfloat16),
                        cat.astype(jnp.bfloat16), (((1,), (0,)), ((), ())),
                        preferred_element_type=jnp.float32) + b0_ref[...]
    u_ref[0] = u

    @pl.when(jnp.logical_and(b == 0, t == 0))
    def _():
        s_ref[...] = jnp.zeros_like(s_ref)
        ss_ref[...] = jnp.zeros_like(ss_ref)

    s_ref[...] += jnp.sum(u, axis=1, keepdims=True)
    ss_ref[...] += jnp.sum(u * u, axis=1, keepdims=True)


def _stage2_body(nt_inv, u_ref, s_ref, ss_ref, g_ref, be_ref, w1_ref, b1_ref,
                 v_ref, s1_ref, ss1_ref):
    b = pl.program_id(0)
    t = pl.program_id(1)
    mean = s_ref[...] * nt_inv
    var = ss_ref[...] * nt_inv - mean * mean
    sc = g_ref[...] / jnp.sqrt(var + 1e-5)
    sh = be_ref[...] - sc * mean
    r = jnp.maximum(sc * u_ref[0] + sh, 0.0)
    v = lax.dot_general(w1_ref[...].astype(jnp.bfloat16),
                        r.astype(jnp.bfloat16), (((1,), (0,)), ((), ())),
                        preferred_element_type=jnp.float32) + b1_ref[...]
    v_ref[0] = v

    @pl.when(jnp.logical_and(b == 0, t == 0))
    def _():
        s1_ref[...] = jnp.zeros_like(s1_ref)
        ss1_ref[...] = jnp.zeros_like(ss1_ref)

    s1_ref[...] += jnp.sum(v, axis=1, keepdims=True)
    ss1_ref[...] += jnp.sum(v * v, axis=1, keepdims=True)


def _stage3_body(nt_inv, v_ref, s_ref, ss_ref, g_ref, be_ref, o_ref):
    mean = s_ref[...] * nt_inv
    var = ss_ref[...] * nt_inv - mean * mean
    sc = g_ref[...] / jnp.sqrt(var + 1e-5)
    sh = be_ref[...] - sc * mean
    o_ref[0] = jnp.maximum(sc * v_ref[0] + sh, 0.0)


def kernel(xyz1, xyz2, points1, points2, w0, b0, g0, be0, w1, b1, g1, be1):
    B, _, N = xyz1.shape
    S = xyz2.shape[2]
    D1 = points1.shape[1]
    D2 = points2.shape[1]
    C0 = w0.shape[0]
    C1 = w1.shape[0]
    nt_inv = 1.0 / (B * N)

    x1p = jnp.pad(xyz1, ((0, 0), (0, 5), (0, 0)))
    x2t = jnp.pad(jnp.swapaxes(xyz2, 1, 2),
                  ((0, 0), (0, 0), (0, 5)))
    KC = 256
    w0p = jnp.pad(w0, ((0, 0), (0, KC - (D1 + D2))))
    col = lambda a: a.reshape(-1, 1)

    TN = 256
    grid = (B, N // TN)
    u, s0, ss0 = pl.pallas_call(
        functools.partial(_stage1_body, nt_inv),
        grid=grid,
        in_specs=[
            pl.BlockSpec((1, 8, TN), lambda b, t: (b, 0, t)),
            pl.BlockSpec((1, S, 8), lambda b, t: (b, 0, 0)),
            pl.BlockSpec((1, D1, TN), lambda b, t: (b, 0, t)),
            pl.BlockSpec((1, D2, S), lambda b, t: (b, 0, 0)),
            pl.BlockSpec((C0, KC), lambda b, t: (0, 0)),
            pl.BlockSpec((C0, 1), lambda b, t: (0, 0)),
        ],
        out_specs=[
            pl.BlockSpec((1, C0, TN), lambda b, t: (b, 0, t)),
            pl.BlockSpec((C0, 1), lambda b, t: (0, 0)),
            pl.BlockSpec((C0, 1), lambda b, t: (0, 0)),
        ],
        out_shape=[
            jax.ShapeDtypeStruct((B, C0, N), jnp.float32),
            jax.ShapeDtypeStruct((C0, 1), jnp.float32),
            jax.ShapeDtypeStruct((C0, 1), jnp.float32),
        ],
    )(x1p, x2t, points1, points2, w0p, col(b0))

    TN2 = 512
    v, s1, ss1 = pl.pallas_call(
        functools.partial(_stage2_body, nt_inv),
        grid=(B, N // TN2),
        in_specs=[
            pl.BlockSpec((1, C0, TN2), lambda b, t: (b, 0, t)),
            pl.BlockSpec((C0, 1), lambda b, t: (0, 0)),
            pl.BlockSpec((C0, 1), lambda b, t: (0, 0)),
            pl.BlockSpec((C0, 1), lambda b, t: (0, 0)),
            pl.BlockSpec((C0, 1), lambda b, t: (0, 0)),
            pl.BlockSpec((C1, C0), lambda b, t: (0, 0)),
            pl.BlockSpec((C1, 1), lambda b, t: (0, 0)),
        ],
        out_specs=[
            pl.BlockSpec((1, C1, TN2), lambda b, t: (b, 0, t)),
            pl.BlockSpec((C1, 1), lambda b, t: (0, 0)),
            pl.BlockSpec((C1, 1), lambda b, t: (0, 0)),
        ],
        out_shape=[
            jax.ShapeDtypeStruct((B, C1, N), jnp.float32),
            jax.ShapeDtypeStruct((C1, 1), jnp.float32),
            jax.ShapeDtypeStruct((C1, 1), jnp.float32),
        ],
    )(u, s0, ss0, col(g0), col(be0), w1, col(b1))

    TN3 = 2048
    out = pl.pallas_call(
        functools.partial(_stage3_body, nt_inv),
        grid=(B, N // TN3),
        in_specs=[
            pl.BlockSpec((1, C1, TN3), lambda b, t: (b, 0, t)),
            pl.BlockSpec((C1, 1), lambda b, t: (0, 0)),
            pl.BlockSpec((C1, 1), lambda b, t: (0, 0)),
            pl.BlockSpec((C1, 1), lambda b, t: (0, 0)),
            pl.BlockSpec((C1, 1), lambda b, t: (0, 0)),
        ],
        out_specs=pl.BlockSpec((1, C1, TN3), lambda b, t: (b, 0, t)),
        out_shape=jax.ShapeDtypeStruct((B, C1, N), jnp.float32),
    )(v, s1, ss1, col(g1), col(be1))
    return out


# speedup vs baseline: 56.2387x; 2.3247x over previous
import functools

import jax
import jax.numpy as jnp
from jax import lax
from jax.experimental import pallas as pl


_HI = jax.lax.Precision.HIGHEST


def _stage1_body(nt_inv, x1_ref, x1b_ref, x2tb_ref, n2_ref, p1_ref, p2h_ref,
                 p2l_ref, w0_ref, b0_ref, u_ref, s_ref, ss_ref):
    b = pl.program_id(0)
    t = pl.program_id(1)
    x1 = x1_ref[0]

    cross = lax.dot_general(x2tb_ref[0], x1b_ref[0], (((1,), (0,)), ((), ())),
                            preferred_element_type=jnp.float32)
    n1 = (x1[0:1, :] * x1[0:1, :] + x1[1:2, :] * x1[1:2, :]
          + x1[2:3, :] * x1[2:3, :])
    d = (-2.0 * cross + n1) + n2_ref[0]

    m0 = jnp.min(d, axis=0, keepdims=True)
    eq0 = d == m0
    d = jnp.where(eq0, jnp.float32(jnp.inf), d)
    m1 = jnp.min(d, axis=0, keepdims=True)
    eq1 = d == m1
    d = jnp.where(eq1, jnp.float32(jnp.inf), d)
    m2 = jnp.min(d, axis=0, keepdims=True)
    eq2 = d == m2

    r0 = 1.0 / (m0 + 1e-8)
    r1 = 1.0 / (m1 + 1e-8)
    r2 = 1.0 / (m2 + 1e-8)
    norm = r0 + r1 + r2

    S = cross.shape[0]
    TN = x1.shape[1]
    zero = jnp.zeros((S, TN), jnp.float32)
    wt = jnp.where(eq0, r0 / norm,
                   jnp.where(eq1, r1 / norm,
                             jnp.where(eq2, r2 / norm, zero)))

    wth = wt.astype(jnp.bfloat16)
    dn = (((1,), (0,)), ((), ()))
    f32 = jnp.float32
    interp = (lax.dot_general(p2h_ref[0], wth, dn, preferred_element_type=f32)
              + lax.dot_general(p2l_ref[0], wth, dn, preferred_element_type=f32))
    TNw = x1.shape[1]
    cat = jnp.concatenate(
        [p1_ref[0], interp,
         jnp.zeros((w0_ref.shape[1] - p1_ref.shape[1] - interp.shape[0], TNw),
                   jnp.float32)], axis=0)
    u = lax.dot_general(w0_ref[...].astype(jnp.bfloat16),
                        cat.astype(jnp.bfloat16), (((1,), (0,)), ((), ())),
                        preferred_element_type=jnp.float32) + b0_ref[...]
    u_ref[0] = u

    @pl.when(jnp.logical_and(b == 0, t == 0))
    def _():
        s_ref[...] = jnp.zeros_like(s_ref)
        ss_ref[...] = jnp.zeros_like(ss_ref)

    s_ref[...] += jnp.sum(u, axis=1, keepdims=True)
    ss_ref[...] += jnp.sum(u * u, axis=1, keepdims=True)


def _stage2_body(nt_inv, u_ref, s_ref, ss_ref, g_ref, be_ref, w1_ref, b1_ref,
                 v_ref, s1_ref, ss1_ref):
    b = pl.program_id(0)
    t = pl.program_id(1)
    mean = s_ref[...] * nt_inv
    var = ss_ref[...] * nt_inv - mean * mean
    sc = g_ref[...] / jnp.sqrt(var + 1e-5)
    sh = be_ref[...] - sc * mean
    r = jnp.maximum(sc * u_ref[0] + sh, 0.0)
    v = lax.dot_general(w1_ref[...].astype(jnp.bfloat16),
                        r.astype(jnp.bfloat16), (((1,), (0,)), ((), ())),
                        preferred_element_type=jnp.float32) + b1_ref[...]
    v_ref[0] = v

    @pl.when(jnp.logical_and(b == 0, t == 0))
    def _():
        s1_ref[...] = jnp.zeros_like(s1_ref)
        ss1_ref[...] = jnp.zeros_like(ss1_ref)

    s1_ref[...] += jnp.sum(v, axis=1, keepdims=True)
    ss1_ref[...] += jnp.sum(v * v, axis=1, keepdims=True)


def _stage3_body(nt_inv, v_ref, s_ref, ss_ref, g_ref, be_ref, o_ref):
    mean = s_ref[...] * nt_inv
    var = ss_ref[...] * nt_inv - mean * mean
    sc = g_ref[...] / jnp.sqrt(var + 1e-5)
    sh = be_ref[...] - sc * mean
    o_ref[0] = jnp.maximum(sc * v_ref[0] + sh, 0.0)


def kernel(xyz1, xyz2, points1, points2, w0, b0, g0, be0, w1, b1, g1, be1):
    B, _, N = xyz1.shape
    S = xyz2.shape[2]
    D1 = points1.shape[1]
    D2 = points2.shape[1]
    C0 = w0.shape[0]
    C1 = w1.shape[0]
    nt_inv = 1.0 / (B * N)

    x1p = jnp.pad(xyz1, ((0, 0), (0, 5), (0, 0)))
    x2ts = jnp.swapaxes(xyz2, 1, 2)
    x1b = jnp.pad(xyz1.astype(jnp.bfloat16),
                  ((0, 0), (0, 125), (0, 0)))
    x2tb = jnp.pad(x2ts.astype(jnp.bfloat16),
                   ((0, 0), (0, 0), (0, 125)))
    n2c = jnp.sum(x2ts * x2ts, axis=2, keepdims=True)
    p2h = points2.astype(jnp.bfloat16)
    p2l = (points2 - p2h.astype(jnp.float32)).astype(jnp.bfloat16)
    KC = 256
    w0p = jnp.pad(w0, ((0, 0), (0, KC - (D1 + D2))))
    col = lambda a: a.reshape(-1, 1)

    TN = 1024
    grid = (B, N // TN)
    u, s0, ss0 = pl.pallas_call(
        functools.partial(_stage1_body, nt_inv),
        grid=grid,
        in_specs=[
            pl.BlockSpec((1, 8, TN), lambda b, t: (b, 0, t)),
            pl.BlockSpec((1, 128, TN), lambda b, t: (b, 0, t)),
            pl.BlockSpec((1, S, 128), lambda b, t: (b, 0, 0)),
            pl.BlockSpec((1, S, 1), lambda b, t: (b, 0, 0)),
            pl.BlockSpec((1, D1, TN), lambda b, t: (b, 0, t)),
            pl.BlockSpec((1, D2, S), lambda b, t: (b, 0, 0)),
            pl.BlockSpec((1, D2, S), lambda b, t: (b, 0, 0)),
            pl.BlockSpec((C0, KC), lambda b, t: (0, 0)),
            pl.BlockSpec((C0, 1), lambda b, t: (0, 0)),
        ],
        out_specs=[
            pl.BlockSpec((1, C0, TN), lambda b, t: (b, 0, t)),
            pl.BlockSpec((C0, 1), lambda b, t: (0, 0)),
            pl.BlockSpec((C0, 1), lambda b, t: (0, 0)),
        ],
        out_shape=[
            jax.ShapeDtypeStruct((B, C0, N), jnp.float32),
            jax.ShapeDtypeStruct((C0, 1), jnp.float32),
            jax.ShapeDtypeStruct((C0, 1), jnp.float32),
        ],
    )(x1p, x1b, x2tb, n2c, points1, p2h, p2l, w0p, col(b0))

    TN2 = 512
    v, s1, ss1 = pl.pallas_call(
        functools.partial(_stage2_body, nt_inv),
        grid=(B, N // TN2),
        in_specs=[
            pl.BlockSpec((1, C0, TN2), lambda b, t: (b, 0, t)),
            pl.BlockSpec((C0, 1), lambda b, t: (0, 0)),
            pl.BlockSpec((C0, 1), lambda b, t: (0, 0)),
            pl.BlockSpec((C0, 1), lambda b, t: (0, 0)),
            pl.BlockSpec((C0, 1), lambda b, t: (0, 0)),
            pl.BlockSpec((C1, C0), lambda b, t: (0, 0)),
            pl.BlockSpec((C1, 1), lambda b, t: (0, 0)),
        ],
        out_specs=[
            pl.BlockSpec((1, C1, TN2), lambda b, t: (b, 0, t)),
            pl.BlockSpec((C1, 1), lambda b, t: (0, 0)),
            pl.BlockSpec((C1, 1), lambda b, t: (0, 0)),
        ],
        out_shape=[
            jax.ShapeDtypeStruct((B, C1, N), jnp.float32),
            jax.ShapeDtypeStruct((C1, 1), jnp.float32),
            jax.ShapeDtypeStruct((C1, 1), jnp.float32),
        ],
    )(u, s0, ss0, col(g0), col(be0), w1, col(b1))

    TN3 = 2048
    out = pl.pallas_call(
        functools.partial(_stage3_body, nt_inv),
        grid=(B, N // TN3),
        in_specs=[
            pl.BlockSpec((1, C1, TN3), lambda b, t: (b, 0, t)),
            pl.BlockSpec((C1, 1), lambda b, t: (0, 0)),
            pl.BlockSpec((C1, 1), lambda b, t: (0, 0)),
            pl.BlockSpec((C1, 1), lambda b, t: (0, 0)),
            pl.BlockSpec((C1, 1), lambda b, t: (0, 0)),
        ],
        out_specs=pl.BlockSpec((1, C1, TN3), lambda b, t: (b, 0, t)),
        out_shape=jax.ShapeDtypeStruct((B, C1, N), jnp.float32),
    )(v, s1, ss1, col(g1), col(be1))
    return out


# speedup vs baseline: 62.4956x; 1.1113x over previous
import functools

import jax
import jax.numpy as jnp
from jax import lax
from jax.experimental import pallas as pl


def _stage1_body(nt_inv, x1_ref, x1b_ref, x2tb_ref, n2_ref, p1_ref, p2h_ref,
                 p2l_ref, w0_ref, b0_ref, u_ref, s_ref, ss_ref):
    b = pl.program_id(0)
    t = pl.program_id(1)
    x1 = x1_ref[0]

    cross = lax.dot_general(x2tb_ref[0], x1b_ref[0], (((1,), (0,)), ((), ())),
                            preferred_element_type=jnp.float32)
    n1 = (x1[0:1, :] * x1[0:1, :] + x1[1:2, :] * x1[1:2, :]
          + x1[2:3, :] * x1[2:3, :])
    d = (-2.0 * cross + n1) + n2_ref[0]

    m0 = jnp.min(d, axis=0, keepdims=True)
    eq0 = d == m0
    d = jnp.where(eq0, jnp.float32(jnp.inf), d)
    m1 = jnp.min(d, axis=0, keepdims=True)
    eq1 = d == m1
    d = jnp.where(eq1, jnp.float32(jnp.inf), d)
    m2 = jnp.min(d, axis=0, keepdims=True)
    eq2 = d == m2

    r0 = 1.0 / (m0 + 1e-8)
    r1 = 1.0 / (m1 + 1e-8)
    r2 = 1.0 / (m2 + 1e-8)
    norm = r0 + r1 + r2

    S = cross.shape[0]
    TN = x1.shape[1]
    zero = jnp.zeros((S, TN), jnp.float32)
    wt = jnp.where(eq0, r0 / norm,
                   jnp.where(eq1, r1 / norm,
                             jnp.where(eq2, r2 / norm, zero)))

    wth = wt.astype(jnp.bfloat16)
    dn = (((1,), (0,)), ((), ()))
    f32 = jnp.float32
    interp = (lax.dot_general(p2h_ref[0], wth, dn, preferred_element_type=f32)
              + lax.dot_general(p2l_ref[0], wth, dn, preferred_element_type=f32))
    TNw = x1.shape[1]
    cat = jnp.concatenate(
        [p1_ref[0], interp,
         jnp.zeros((w0_ref.shape[1] - p1_ref.shape[1] - interp.shape[0], TNw),
                   jnp.float32)], axis=0)
    u = lax.dot_general(w0_ref[...].astype(jnp.bfloat16),
                        cat.astype(jnp.bfloat16), (((1,), (0,)), ((), ())),
                        preferred_element_type=jnp.float32) + b0_ref[...]
    u_ref[0] = u

    @pl.when(jnp.logical_and(b == 0, t == 0))
    def _():
        s_ref[...] = jnp.zeros_like(s_ref)
        ss_ref[...] = jnp.zeros_like(ss_ref)

    s_ref[...] += jnp.sum(u, axis=1, keepdims=True)
    ss_ref[...] += jnp.sum(u * u, axis=1, keepdims=True)


def _stage2_body(nt_inv, u_ref, s_ref, ss_ref, g_ref, be_ref, w1_ref, b1_ref,
                 v_ref, s1_ref, ss1_ref):
    b = pl.program_id(0)
    t = pl.program_id(1)
    mean = s_ref[...] * nt_inv
    var = ss_ref[...] * nt_inv - mean * mean
    sc = g_ref[...] / jnp.sqrt(var + 1e-5)
    sh = be_ref[...] - sc * mean
    r = jnp.maximum(sc * u_ref[0] + sh, 0.0)
    v = lax.dot_general(w1_ref[...].astype(jnp.bfloat16),
                        r.astype(jnp.bfloat16), (((1,), (0,)), ((), ())),
                        preferred_element_type=jnp.float32) + b1_ref[...]
    v_ref[0] = v

    @pl.when(jnp.logical_and(b == 0, t == 0))
    def _():
        s1_ref[...] = jnp.zeros_like(s1_ref)
        ss1_ref[...] = jnp.zeros_like(ss1_ref)

    s1_ref[...] += jnp.sum(v, axis=1, keepdims=True)
    ss1_ref[...] += jnp.sum(v * v, axis=1, keepdims=True)


def _stage3_body(nt_inv, v_ref, s_ref, ss_ref, g_ref, be_ref, o_ref):
    mean = s_ref[...] * nt_inv
    var = ss_ref[...] * nt_inv - mean * mean
    sc = g_ref[...] / jnp.sqrt(var + 1e-5)
    sh = be_ref[...] - sc * mean
    o_ref[0] = jnp.maximum(sc * v_ref[0] + sh, 0.0)


def kernel(xyz1, xyz2, points1, points2, w0, b0, g0, be0, w1, b1, g1, be1):
    B, _, N = xyz1.shape
    S = xyz2.shape[2]
    D1 = points1.shape[1]
    D2 = points2.shape[1]
    C0 = w0.shape[0]
    C1 = w1.shape[0]
    nt_inv = 1.0 / (B * N)

    x1p = jnp.pad(xyz1, ((0, 0), (0, 5), (0, 0)))
    x2ts = jnp.swapaxes(xyz2, 1, 2)
    x1b = jnp.pad(xyz1.astype(jnp.bfloat16),
                  ((0, 0), (0, 125), (0, 0)))
    x2tb = jnp.pad(x2ts.astype(jnp.bfloat16),
                   ((0, 0), (0, 0), (0, 125)))
    n2c = jnp.sum(x2ts * x2ts, axis=2, keepdims=True)
    p2h = points2.astype(jnp.bfloat16)
    p2l = (points2 - p2h.astype(jnp.float32)).astype(jnp.bfloat16)
    KC = 256
    w0p = jnp.pad(w0, ((0, 0), (0, KC - (D1 + D2))))
    col = lambda a: a.reshape(-1, 1)

    TN = 1024
    grid = (B, N // TN)
    u, s0, ss0 = pl.pallas_call(
        functools.partial(_stage1_body, nt_inv),
        grid=grid,
        in_specs=[
            pl.BlockSpec((1, 8, TN), lambda b, t: (b, 0, t)),
            pl.BlockSpec((1, 128, TN), lambda b, t: (b, 0, t)),
            pl.BlockSpec((1, S, 128), lambda b, t: (b, 0, 0)),
            pl.BlockSpec((1, S, 1), lambda b, t: (b, 0, 0)),
            pl.BlockSpec((1, D1, TN), lambda b, t: (b, 0, t)),
            pl.BlockSpec((1, D2, S), lambda b, t: (b, 0, 0)),
            pl.BlockSpec((1, D2, S), lambda b, t: (b, 0, 0)),
            pl.BlockSpec((C0, KC), lambda b, t: (0, 0)),
            pl.BlockSpec((C0, 1), lambda b, t: (0, 0)),
        ],
        out_specs=[
            pl.BlockSpec((1, C0, TN), lambda b, t: (b, 0, t)),
            pl.BlockSpec((C0, 1), lambda b, t: (0, 0)),
            pl.BlockSpec((C0, 1), lambda b, t: (0, 0)),
        ],
        out_shape=[
            jax.ShapeDtypeStruct((B, C0, N), jnp.float32),
            jax.ShapeDtypeStruct((C0, 1), jnp.float32),
            jax.ShapeDtypeStruct((C0, 1), jnp.float32),
        ],
    )(x1p, x1b, x2tb, n2c, points1, p2h, p2l, w0p, col(b0))

    TN2 = 2048
    v, s1, ss1 = pl.pallas_call(
        functools.partial(_stage2_body, nt_inv),
        grid=(B, N // TN2),
        in_specs=[
            pl.BlockSpec((1, C0, TN2), lambda b, t: (b, 0, t)),
            pl.BlockSpec((C0, 1), lambda b, t: (0, 0)),
            pl.BlockSpec((C0, 1), lambda b, t: (0, 0)),
            pl.BlockSpec((C0, 1), lambda b, t: (0, 0)),
            pl.BlockSpec((C0, 1), lambda b, t: (0, 0)),
            pl.BlockSpec((C1, C0), lambda b, t: (0, 0)),
            pl.BlockSpec((C1, 1), lambda b, t: (0, 0)),
        ],
        out_specs=[
            pl.BlockSpec((1, C1, TN2), lambda b, t: (b, 0, t)),
            pl.BlockSpec((C1, 1), lambda b, t: (0, 0)),
            pl.BlockSpec((C1, 1), lambda b, t: (0, 0)),
        ],
        out_shape=[
            jax.ShapeDtypeStruct((B, C1, N), jnp.float32),
            jax.ShapeDtypeStruct((C1, 1), jnp.float32),
            jax.ShapeDtypeStruct((C1, 1), jnp.float32),
        ],
    )(u, s0, ss0, col(g0), col(be0), w1, col(b1))

    TN3 = 2048
    out = pl.pallas_call(
        functools.partial(_stage3_body, nt_inv),
        grid=(B, N // TN3),
        in_specs=[
            pl.BlockSpec((1, C1, TN3), lambda b, t: (b, 0, t)),
            pl.BlockSpec((C1, 1), lambda b, t: (0, 0)),
            pl.BlockSpec((C1, 1), lambda b, t: (0, 0)),
            pl.BlockSpec((C1, 1), lambda b, t: (0, 0)),
            pl.BlockSpec((C1, 1), lambda b, t: (0, 0)),
        ],
        out_specs=pl.BlockSpec((1, C1, TN3), lambda b, t: (b, 0, t)),
        out_shape=jax.ShapeDtypeStruct((B, C1, N), jnp.float32),
    )(v, s1, ss1, col(g1), col(be1))
    return out
